# baseline (speedup 1.0000x reference)
import functools

import jax
import jax.numpy as jnp
from jax import lax
from jax.experimental import pallas as pl
from jax.experimental.pallas import tpu as pltpu
from jax.experimental.pallas import tpu_sc as plsc

_B = 4
_N = 8192
_NQ = 1024
_K = 16
_L = 16
_NC = 2
_NS = 16
_NW = _NC * _NS
_RPW = _B * _NQ // _NW
_RG = 4
_NCHUNK = _N // _L


def _sc_knn_body(xs, ys, zs, qxe, qye, qze, out, xv, yv, zv, qxv, qyv, qzv, ov):
    cid = lax.axis_index("c")
    sid = lax.axis_index("s")
    wid = sid * _NC + cid
    b = wid // (_NW // _B)
    q0 = (wid % (_NW // _B)) * _RPW
    roff = b * _NQ + q0

    pltpu.sync_copy(xs.at[pl.ds(b * _N, _N)], xv)
    pltpu.sync_copy(ys.at[pl.ds(b * _N, _N)], yv)
    pltpu.sync_copy(zs.at[pl.ds(b * _N, _N)], zv)
    pltpu.sync_copy(qxe.at[pl.ds(roff * _L, _RPW * _L)], qxv)
    pltpu.sync_copy(qye.at[pl.ds(roff * _L, _RPW * _L)], qyv)
    pltpu.sync_copy(qze.at[pl.ds(roff * _L, _RPW * _L)], qzv)

    iota16 = lax.iota(jnp.int32, _L)
    inf16 = jnp.full((_L,), jnp.inf, jnp.float32)
    bigi = jnp.int32(2 ** 30)

    def group_body(g, _):
        qb = []
        for j in range(_RG):
            base = (g * _RG + j) * _L
            qb.append((qxv[pl.ds(base, _L)],
                       qyv[pl.ds(base, _L)],
                       qzv[pl.ds(base, _L)]))

        def chunk_body(c, carry):
            cbase = c * _L
            cx = xv[pl.ds(cbase, _L)]
            cy = yv[pl.ds(cbase, _L)]
            cz = zv[pl.ds(cbase, _L)]
            idx = iota16 + cbase
            st = []
            for j in range(_RG):
                bd, bi, thr = carry[3 * j], carry[3 * j + 1], carry[3 * j + 2]
                dx = qb[j][0] - cx
                dy = qb[j][1] - cy
                dz = qb[j][2] - cz
                d2 = (dx * dx + dy * dy) + dz * dz

                def _merge(d2, idx, bd, bi):
                    sd, si = plsc.sort_key_val(d2, idx)
                    pick = (sd < bd) | ((sd == bd) & (si < bi))
                    md = jnp.where(pick, sd, bd)
                    mi = jnp.where(pick, si, bi)
                    md, mi = plsc.sort_key_val(md, mi, descending=True)
                    return md, mi, jnp.broadcast_to(jnp.max(md), (_L,))

                hit = jnp.any(d2 < thr)
                bd, bi, thr = lax.cond(
                    hit, _merge, lambda d2, idx, bd, bi: (bd, bi, thr),
                    d2, idx, bd, bi)
                st.extend([bd, bi, thr])
            return tuple(st)

        init = []
        for j in range(_RG):
            init.extend([inf16, iota16, inf16])
        res = lax.fori_loop(0, _NCHUNK, chunk_body, tuple(init))

        cds = [res[3 * j] for j in range(_RG)]
        cis = [res[3 * j + 1] for j in range(_RG)]
        outs = [jnp.zeros((_L,), jnp.int32) for _ in range(_RG)]
        for k in range(_K):
            for j in range(_RG):
                mind = jnp.min(cds[j])
                eq = cds[j] == mind
                mini = jnp.min(jnp.where(eq, cis[j], bigi))
                outs[j] = jnp.where(iota16 == k, mini, outs[j])
                cds[j] = jnp.where(eq & (cis[j] == mini), jnp.inf, cds[j])
        for j in range(_RG):
            ov[pl.ds((g * _RG + j) * _K, _K)] = outs[j]
        return 0

    lax.fori_loop(0, _RPW // _RG, group_body, 0)
    pltpu.sync_copy(ov, out.at[pl.ds(roff * _K, _RPW * _K)])


_sc_knn = functools.partial(
    pl.kernel,
    out_type=jax.ShapeDtypeStruct((_B * _NQ * _K,), jnp.int32),
    mesh=plsc.VectorSubcoreMesh(core_axis_name="c", subcore_axis_name="s",
                                num_cores=_NC, num_subcores=_NS),
    compiler_params=pltpu.CompilerParams(needs_layout_passes=False),
    scratch_types=[
        pltpu.VMEM((_N,), jnp.float32),
        pltpu.VMEM((_N,), jnp.float32),
        pltpu.VMEM((_N,), jnp.float32),
        pltpu.VMEM((_RPW * _L,), jnp.float32),
        pltpu.VMEM((_RPW * _L,), jnp.float32),
        pltpu.VMEM((_RPW * _L,), jnp.float32),
        pltpu.VMEM((_RPW * _K,), jnp.int32),
    ],
)(_sc_knn_body)


def kernel(xyz):
    b, n, _ = xyz.shape
    stride = n // _NQ
    queries = xyz[:, ::stride, :]

    xs = xyz[..., 0].reshape(-1)
    ys = xyz[..., 1].reshape(-1)
    zs = xyz[..., 2].reshape(-1)
    qe = jnp.broadcast_to(queries[:, :, None, :], (b, _NQ, _L, 3))
    qxe = qe[..., 0].reshape(-1)
    qye = qe[..., 1].reshape(-1)
    qze = qe[..., 2].reshape(-1)

    flat = _sc_knn(xs, ys, zs, qxe, qye, qze)
    knn_idx = flat.reshape(b, _NQ, _K)
    return knn_idx.astype(jnp.int64)[..., None], queries


# speedup vs baseline: 2.6185x; 2.6185x over previous
import functools

import jax
import jax.numpy as jnp
from jax import lax
from jax.experimental import pallas as pl
from jax.experimental.pallas import tpu as pltpu
from jax.experimental.pallas import tpu_sc as plsc

_B = 4
_N = 8192
_NQ = 1024
_K = 16
_L = 16
_NC = 2
_NS = 16
_NW = _NC * _NS
_RPW = _B * _NQ // _NW
_RG = 4
_NCHUNK = _N // _L


def _sc_knn_body(xs, ys, zs, qxe, qye, qze, out, xv, yv, zv, qxv, qyv, qzv, ov):
    cid = lax.axis_index("c")
    sid = lax.axis_index("s")
    wid = sid * _NC + cid
    b = wid // (_NW // _B)
    q0 = (wid % (_NW // _B)) * _RPW
    roff = b * _NQ + q0

    pltpu.sync_copy(xs.at[pl.ds(b * _N, _N)], xv)
    pltpu.sync_copy(ys.at[pl.ds(b * _N, _N)], yv)
    pltpu.sync_copy(zs.at[pl.ds(b * _N, _N)], zv)
    pltpu.sync_copy(qxe.at[pl.ds(roff * _L, _RPW * _L)], qxv)
    pltpu.sync_copy(qye.at[pl.ds(roff * _L, _RPW * _L)], qyv)
    pltpu.sync_copy(qze.at[pl.ds(roff * _L, _RPW * _L)], qzv)

    iota16 = lax.iota(jnp.int32, _L)
    inf16 = jnp.full((_L,), jnp.inf, jnp.float32)
    bigi = jnp.int32(2 ** 30)

    def group_body(g, _):
        qb = []
        for j in range(_RG):
            base = (g * _RG + j) * _L
            qb.append((qxv[pl.ds(base, _L)],
                       qyv[pl.ds(base, _L)],
                       qzv[pl.ds(base, _L)]))

        def chunk_body(c, carry):
            st = list(carry)
            for u in range(2):
                cbase = (2 * c + u) * _L
                cx = xv[pl.ds(cbase, _L)]
                cy = yv[pl.ds(cbase, _L)]
                cz = zv[pl.ds(cbase, _L)]
                idx = iota16 + cbase
                for j in range(_RG):
                    bd, bi = st[2 * j], st[2 * j + 1]
                    dx = qb[j][0] - cx
                    dy = qb[j][1] - cy
                    dz = qb[j][2] - cz
                    d2 = (dx * dx + dy * dy) + dz * dz
                    sd, si = plsc.sort_key_val(d2, idx)
                    pick = (sd < bd) | ((sd == bd) & (si < bi))
                    md = jnp.where(pick, sd, bd)
                    mi = jnp.where(pick, si, bi)
                    md, mi = plsc.sort_key_val(md, mi, descending=True)
                    st[2 * j] = md
                    st[2 * j + 1] = mi
            return tuple(st)

        init = []
        for j in range(_RG):
            init.extend([inf16, iota16])
        res = lax.fori_loop(0, _NCHUNK // 2, chunk_body, tuple(init))

        cds = [res[2 * j] for j in range(_RG)]
        cis = [res[2 * j + 1] for j in range(_RG)]
        outs = [jnp.zeros((_L,), jnp.int32) for _ in range(_RG)]
        for k in range(_K):
            for j in range(_RG):
                mind = jnp.min(cds[j])
                eq = cds[j] == mind
                mini = jnp.min(jnp.where(eq, cis[j], bigi))
                outs[j] = jnp.where(iota16 == k, mini, outs[j])
                cds[j] = jnp.where(eq & (cis[j] == mini), jnp.inf, cds[j])
        for j in range(_RG):
            ov[pl.ds((g * _RG + j) * _K, _K)] = outs[j]
        return 0

    lax.fori_loop(0, _RPW // _RG, group_body, 0)
    pltpu.sync_copy(ov, out.at[pl.ds(roff * _K, _RPW * _K)])


_sc_knn = functools.partial(
    pl.kernel,
    out_type=jax.ShapeDtypeStruct((_B * _NQ * _K,), jnp.int32),
    mesh=plsc.VectorSubcoreMesh(core_axis_name="c", subcore_axis_name="s",
                                num_cores=_NC, num_subcores=_NS),
    compiler_params=pltpu.CompilerParams(needs_layout_passes=False),
    scratch_types=[
        pltpu.VMEM((_N,), jnp.float32),
        pltpu.VMEM((_N,), jnp.float32),
        pltpu.VMEM((_N,), jnp.float32),
        pltpu.VMEM((_RPW * _L,), jnp.float32),
        pltpu.VMEM((_RPW * _L,), jnp.float32),
        pltpu.VMEM((_RPW * _L,), jnp.float32),
        pltpu.VMEM((_RPW * _K,), jnp.int32),
    ],
)(_sc_knn_body)


def kernel(xyz):
    b, n, _ = xyz.shape
    stride = n // _NQ
    queries = xyz[:, ::stride, :]

    xs = xyz[..., 0].reshape(-1)
    ys = xyz[..., 1].reshape(-1)
    zs = xyz[..., 2].reshape(-1)
    qe = jnp.broadcast_to(queries[:, :, None, :], (b, _NQ, _L, 3))
    qxe = qe[..., 0].reshape(-1)
    qye = qe[..., 1].reshape(-1)
    qze = qe[..., 2].reshape(-1)

    flat = _sc_knn(xs, ys, zs, qxe, qye, qze)
    knn_idx = flat.reshape(b, _NQ, _K)
    return knn_idx.astype(jnp.int64)[..., None], queries


# speedup vs baseline: 2.9736x; 1.1356x over previous
import functools

import jax
import jax.numpy as jnp
from jax import lax
from jax.experimental import pallas as pl
from jax.experimental.pallas import tpu as pltpu
from jax.experimental.pallas import tpu_sc as plsc

_B = 4
_N = 8192
_NQ = 1024
_K = 16
_L = 16
_NC = 2
_NS = 16
_NW = _NC * _NS
_RPW = _B * _NQ // _NW
_RG = 4
_NCHUNK = _N // _L


def _sc_knn_body(xs, ys, zs, qxe, qye, qze, out, xv, yv, zv, qxv, qyv, qzv, ov):
    cid = lax.axis_index("c")
    sid = lax.axis_index("s")
    wid = sid * _NC + cid
    b = wid // (_NW // _B)
    q0 = (wid % (_NW // _B)) * _RPW
    roff = b * _NQ + q0

    pltpu.sync_copy(xs.at[pl.ds(b * _N, _N)], xv)
    pltpu.sync_copy(ys.at[pl.ds(b * _N, _N)], yv)
    pltpu.sync_copy(zs.at[pl.ds(b * _N, _N)], zv)
    pltpu.sync_copy(qxe.at[pl.ds(roff * _L, _RPW * _L)], qxv)
    pltpu.sync_copy(qye.at[pl.ds(roff * _L, _RPW * _L)], qyv)
    pltpu.sync_copy(qze.at[pl.ds(roff * _L, _RPW * _L)], qzv)

    iota16 = lax.iota(jnp.int32, _L)
    inf16 = jnp.full((_L,), jnp.inf, jnp.float32)
    bigi = jnp.int32(2 ** 30)

    def group_body(g, _):
        qb = []
        for j in range(_RG):
            base = (g * _RG + j) * _L
            qb.append((qxv[pl.ds(base, _L)],
                       qyv[pl.ds(base, _L)],
                       qzv[pl.ds(base, _L)]))

        def chunk_body(c, carry):
            cbase = c * _L
            cx = xv[pl.ds(cbase, _L)]
            cy = yv[pl.ds(cbase, _L)]
            cz = zv[pl.ds(cbase, _L)]
            idx = iota16 + cbase
            st = []
            for j in range(_RG):
                bd, bi = carry[2 * j], carry[2 * j + 1]
                dx = qb[j][0] - cx
                dy = qb[j][1] - cy
                dz = qb[j][2] - cz
                d2 = (dx * dx + dy * dy) + dz * dz
                sd, si = plsc.sort_key_val(d2, idx)
                pick = sd < bd
                md = jnp.where(pick, sd, bd)
                mi = jnp.where(pick, si, bi)
                md, mi = plsc.sort_key_val(md, mi, descending=True)
                st.extend([md, mi])
            return tuple(st)

        init = []
        for j in range(_RG):
            init.extend([inf16, iota16])
        res = lax.fori_loop(0, _NCHUNK, chunk_body, tuple(init))

        cds = [res[2 * j] for j in range(_RG)]
        cis = [res[2 * j + 1] for j in range(_RG)]
        outs = [jnp.zeros((_L,), jnp.int32) for _ in range(_RG)]
        for k in range(_K):
            for j in range(_RG):
                mind = jnp.min(cds[j])
                eq = cds[j] == mind
                mini = jnp.min(jnp.where(eq, cis[j], bigi))
                outs[j] = jnp.where(iota16 == k, mini, outs[j])
                cds[j] = jnp.where(eq & (cis[j] == mini), jnp.inf, cds[j])
        for j in range(_RG):
            ov[pl.ds((g * _RG + j) * _K, _K)] = outs[j]
        return 0

    lax.fori_loop(0, _RPW // _RG, group_body, 0)
    pltpu.sync_copy(ov, out.at[pl.ds(roff * _K, _RPW * _K)])


_sc_knn = functools.partial(
    pl.kernel,
    out_type=jax.ShapeDtypeStruct((_B * _NQ * _K,), jnp.int32),
    mesh=plsc.VectorSubcoreMesh(core_axis_name="c", subcore_axis_name="s",
                                num_cores=_NC, num_subcores=_NS),
    compiler_params=pltpu.CompilerParams(needs_layout_passes=False),
    scratch_types=[
        pltpu.VMEM((_N,), jnp.float32),
        pltpu.VMEM((_N,), jnp.float32),
        pltpu.VMEM((_N,), jnp.float32),
        pltpu.VMEM((_RPW * _L,), jnp.float32),
        pltpu.VMEM((_RPW * _L,), jnp.float32),
        pltpu.VMEM((_RPW * _L,), jnp.float32),
        pltpu.VMEM((_RPW * _K,), jnp.int32),
    ],
)(_sc_knn_body)


def kernel(xyz):
    b, n, _ = xyz.shape
    stride = n // _NQ
    queries = xyz[:, ::stride, :]

    xs = xyz[..., 0].reshape(-1)
    ys = xyz[..., 1].reshape(-1)
    zs = xyz[..., 2].reshape(-1)
    qe = jnp.broadcast_to(queries[:, :, None, :], (b, _NQ, _L, 3))
    qxe = qe[..., 0].reshape(-1)
    qye = qe[..., 1].reshape(-1)
    qze = qe[..., 2].reshape(-1)

    flat = _sc_knn(xs, ys, zs, qxe, qye, qze)
    knn_idx = flat.reshape(b, _NQ, _K)
    return knn_idx.astype(jnp.int64)[..., None], queries


# speedup vs baseline: 3.0164x; 1.0144x over previous
import functools

import jax
import jax.numpy as jnp
from jax import lax
from jax.experimental import pallas as pl
from jax.experimental.pallas import tpu as pltpu
from jax.experimental.pallas import tpu_sc as plsc

_B = 4
_N = 8192
_NQ = 1024
_K = 16
_L = 16
_NC = 2
_NS = 16
_NW = _NC * _NS
_RPW = _B * _NQ // _NW
_RG = 8
_NCHUNK = _N // _L


def _sc_knn_body(xs, ys, zs, qxe, qye, qze, out, xv, yv, zv, qxv, qyv, qzv, ov):
    cid = lax.axis_index("c")
    sid = lax.axis_index("s")
    wid = sid * _NC + cid
    b = wid // (_NW // _B)
    q0 = (wid % (_NW // _B)) * _RPW
    roff = b * _NQ + q0

    pltpu.sync_copy(xs.at[pl.ds(b * _N, _N)], xv)
    pltpu.sync_copy(ys.at[pl.ds(b * _N, _N)], yv)
    pltpu.sync_copy(zs.at[pl.ds(b * _N, _N)], zv)
    pltpu.sync_copy(qxe.at[pl.ds(roff * _L, _RPW * _L)], qxv)
    pltpu.sync_copy(qye.at[pl.ds(roff * _L, _RPW * _L)], qyv)
    pltpu.sync_copy(qze.at[pl.ds(roff * _L, _RPW * _L)], qzv)

    iota16 = lax.iota(jnp.int32, _L)
    inf16 = jnp.full((_L,), jnp.inf, jnp.float32)
    bigi = jnp.int32(2 ** 30)

    def group_body(g, _):
        qb = []
        for j in range(_RG):
            base = (g * _RG + j) * _L
            qb.append((qxv[pl.ds(base, _L)],
                       qyv[pl.ds(base, _L)],
                       qzv[pl.ds(base, _L)]))

        def chunk_body(c, carry):
            cbase = c * _L
            cx = xv[pl.ds(cbase, _L)]
            cy = yv[pl.ds(cbase, _L)]
            cz = zv[pl.ds(cbase, _L)]
            idx = iota16 + cbase
            st = []
            for j in range(_RG):
                bd, bi = carry[2 * j], carry[2 * j + 1]
                dx = qb[j][0] - cx
                dy = qb[j][1] - cy
                dz = qb[j][2] - cz
                d2 = (dx * dx + dy * dy) + dz * dz
                sd, si = plsc.sort_key_val(d2, idx)
                pick = sd < bd
                md = jnp.where(pick, sd, bd)
                mi = jnp.where(pick, si, bi)
                md, mi = plsc.sort_key_val(md, mi, descending=True)
                st.extend([md, mi])
            return tuple(st)

        init = []
        for j in range(_RG):
            init.extend([inf16, iota16])
        res = lax.fori_loop(0, _NCHUNK, chunk_body, tuple(init))

        cds = [res[2 * j] for j in range(_RG)]
        cis = [res[2 * j + 1] for j in range(_RG)]
        outs = [jnp.zeros((_L,), jnp.int32) for _ in range(_RG)]
        for k in range(_K):
            for j in range(_RG):
                mind = jnp.min(cds[j])
                eq = cds[j] == mind
                mini = jnp.min(jnp.where(eq, cis[j], bigi))
                outs[j] = jnp.where(iota16 == k, mini, outs[j])
                cds[j] = jnp.where(eq & (cis[j] == mini), jnp.inf, cds[j])
        for j in range(_RG):
            ov[pl.ds((g * _RG + j) * _K, _K)] = outs[j]
        return 0

    lax.fori_loop(0, _RPW // _RG, group_body, 0)
    pltpu.sync_copy(ov, out.at[pl.ds(roff * _K, _RPW * _K)])


_sc_knn = functools.partial(
    pl.kernel,
    out_type=jax.ShapeDtypeStruct((_B * _NQ * _K,), jnp.int32),
    mesh=plsc.VectorSubcoreMesh(core_axis_name="c", subcore_axis_name="s",
                                num_cores=_NC, num_subcores=_NS),
    compiler_params=pltpu.CompilerParams(needs_layout_passes=False),
    scratch_types=[
        pltpu.VMEM((_N,), jnp.float32),
        pltpu.VMEM((_N,), jnp.float32),
        pltpu.VMEM((_N,), jnp.float32),
        pltpu.VMEM((_RPW * _L,), jnp.float32),
        pltpu.VMEM((_RPW * _L,), jnp.float32),
        pltpu.VMEM((_RPW * _L,), jnp.float32),
        pltpu.VMEM((_RPW * _K,), jnp.int32),
    ],
)(_sc_knn_body)


def kernel(xyz):
    b, n, _ = xyz.shape
    stride = n // _NQ
    queries = xyz[:, ::stride, :]

    xs = xyz[..., 0].reshape(-1)
    ys = xyz[..., 1].reshape(-1)
    zs = xyz[..., 2].reshape(-1)
    qe = jnp.broadcast_to(queries[:, :, None, :], (b, _NQ, _L, 3))
    qxe = qe[..., 0].reshape(-1)
    qye = qe[..., 1].reshape(-1)
    qze = qe[..., 2].reshape(-1)

    flat = _sc_knn(xs, ys, zs, qxe, qye, qze)
    knn_idx = flat.reshape(b, _NQ, _K)
    return knn_idx.astype(jnp.int64)[..., None], queries
